# baseline (speedup 1.0000x reference)
import functools
import math

import numpy as np

import jax
import jax.numpy as jnp
from jax import lax
from jax.experimental import pallas as pl
from jax.experimental.pallas import tpu as pltpu
from jax.experimental.pallas import tpu_sc as plsc

_K = 2
_NUM_EXPERTS = 64
_NUM_BINS = 10
_EXP_BASE = 2.0
_ALIGNMENT = 64
_LANES = 16

_DNUMS = lax.GatherDimensionNumbers(
    offset_dims=(), collapsed_slice_dims=(0,), start_index_map=(0,))


def _bin_edges(total_tokens):
    start = np.float32(math.ceil(total_tokens / _NUM_EXPERTS))
    stop = np.float32(total_tokens)
    widths = np.power(np.float32(_EXP_BASE),
                      np.arange(0, _NUM_BINS, dtype=np.float32),
                      dtype=np.float32)
    normalized = (widths / np.sum(widths)).astype(np.float32)
    edges = np.cumsum(normalized, dtype=np.float32)
    edges = (start + (stop - start) * edges).astype(np.float32)
    return (np.ceil(edges / _ALIGNMENT) * _ALIGNMENT).astype(np.int32)


def _shuffle(x, idx):
    return lax.gather(x, idx[:, None], _DNUMS, slice_sizes=(1,),
                      mode=lax.GatherScatterMode.PROMISE_IN_BOUNDS)


def _make_bucketize(bins16):
    mesh = plsc.ScalarSubcoreMesh(axis_name="c", num_cores=1)

    edges = [int(v) for v in bins16[:_NUM_BINS]]

    def _search(cap, lo, hi):
        if lo == hi:
            return jnp.int32(edges[lo])
        mid = (lo + hi) // 2
        return jnp.where(jnp.int32(edges[mid]) >= cap,
                         _search(cap, lo, mid), _search(cap, mid + 1, hi))

    @functools.partial(
        pl.kernel,
        out_type=jax.ShapeDtypeStruct((1,), jnp.int32),
        mesh=mesh,
        scratch_types=[pltpu.SMEM((1,), jnp.int32)],
        compiler_params=pltpu.CompilerParams(skip_device_barrier=True),
    )
    def _bucketize_sc(cap_hbm, out_hbm, cap_s):
        pltpu.sync_copy(cap_hbm, cap_s)
        cap_s[0] = _search(cap_s[0], 0, _NUM_BINS - 1)
        pltpu.sync_copy(cap_s, out_hbm)

    return _bucketize_sc


def kernel(gate_output, capacity):
    total_tokens = _K * gate_output.shape[0]
    bins = _bin_edges(total_tokens)
    bins16 = np.concatenate(
        [bins, np.broadcast_to(bins[-1:], (_LANES - _NUM_BINS,))])
    return _make_bucketize(bins16)(capacity.astype(jnp.int32))


# speedup vs baseline: 1.0029x; 1.0029x over previous
import functools
import math

import numpy as np

import jax
import jax.numpy as jnp
from jax.experimental import pallas as pl
from jax.experimental.pallas import tpu as pltpu
from jax.experimental.pallas import tpu_sc as plsc

_K = 2
_NUM_EXPERTS = 64
_NUM_BINS = 10
_EXP_BASE = 2.0
_ALIGNMENT = 64


def _bin_edges(total_tokens):
    start = np.float32(math.ceil(total_tokens / _NUM_EXPERTS))
    stop = np.float32(total_tokens)
    widths = np.power(np.float32(_EXP_BASE),
                      np.arange(0, _NUM_BINS, dtype=np.float32),
                      dtype=np.float32)
    normalized = (widths / np.sum(widths)).astype(np.float32)
    edges = np.cumsum(normalized, dtype=np.float32)
    edges = (start + (stop - start) * edges).astype(np.float32)
    return (np.ceil(edges / _ALIGNMENT) * _ALIGNMENT).astype(np.int32)


def _make_bucketize(bins):
    edges = [int(v) for v in bins]

    def _search(cap, lo, hi):
        if lo == hi:
            return jnp.int32(edges[lo])
        mid = (lo + hi) // 2
        return jnp.where(jnp.int32(edges[mid]) >= cap,
                         _search(cap, lo, mid), _search(cap, mid + 1, hi))

    @functools.partial(
        pl.kernel,
        out_type=jax.ShapeDtypeStruct((1,), jnp.int32),
        mesh=plsc.ScalarSubcoreMesh(axis_name="c", num_cores=1),
        scratch_types=[pltpu.SMEM((1,), jnp.int32)],
    )
    def _bucketize_sc(cap_hbm, out_hbm, cap_s):
        pltpu.sync_copy(cap_hbm, cap_s)
        cap_s[0] = _search(cap_s[0], 0, _NUM_BINS - 1)
        pltpu.sync_copy(cap_s, out_hbm)

    return _bucketize_sc


def kernel(gate_output, capacity):
    total_tokens = _K * gate_output.shape[0]
    return _make_bucketize(_bin_edges(total_tokens))(capacity.astype(jnp.int32))
